# Initial kernel scaffold; baseline (speedup 1.0000x reference)
import jax
import jax.numpy as jnp
from jax.experimental import pallas as pl


def kernel(user_idx, item_idx, fav_subjects, book_subjects, subj_emb, attn_weight, attn_bias, user_bias, item_bias, global_bias):
    raise NotImplementedError("write your pallas kernel here")


# speedup vs baseline: 4.6644x; 4.6644x over previous
import functools

import jax
import jax.numpy as jnp
from jax import lax
from jax.experimental import pallas as pl
from jax.experimental.pallas import tpu as pltpu
from jax.experimental.pallas import tpu_sc as plsc

PAD_IDX = 0
NEG_INF = -1e9

NC = 2
NS = 16
NW = NC * NS

GROUPS_PER_STEP = 8
CHUNK = GROUPS_PER_STEP * 128


def _sc_gather_kernel(n_rows, n_steps, bias_groups, D,
                      table, fidx, bidx, uidx, iidx, ubias, ibias,
                      rows_f, rows_b, ub_out, ib_out,
                      idx_v, rows_v, bias_v, sem):
    wid = lax.axis_index("s") * NC + lax.axis_index("c")
    rows_per_w = n_rows // NW
    idxrows_per_w = rows_per_w // 128

    for idx_hbm, out_hbm in ((fidx, rows_f), (bidx, rows_b)):
        def step(s, carry, idx_hbm=idx_hbm, out_hbm=out_hbm):
            row_base = wid * rows_per_w + s * CHUNK
            irow = wid * idxrows_per_w + s * GROUPS_PER_STEP
            pltpu.sync_copy(idx_hbm.at[pl.ds(irow, GROUPS_PER_STEP)], idx_v)
            handles = [
                pltpu.async_copy(table.at[idx_v.at[j]],
                                 rows_v.at[pl.ds(j * 128, 128)], sem)
                for j in range(GROUPS_PER_STEP)
            ]
            for h in handles:
                h.wait()
            pltpu.sync_copy(rows_v, out_hbm.at[pl.ds(row_base, CHUNK)])
            return carry

        lax.fori_loop(0, n_steps, step, 0)

    half = NW // 2
    for active, bidx_hbm, btab, bout in ((wid < half, uidx, ubias, ub_out),
                                         (wid >= half, iidx, ibias, ib_out)):
        @pl.when(active)
        def _(bidx_hbm=bidx_hbm, btab=btab, bout=bout):
            lane = lax.rem(wid, half)
            pltpu.sync_copy(
                bidx_hbm.at[pl.ds(lane * bias_groups, bias_groups)],
                idx_v.at[pl.ds(0, bias_groups)])
            handles = [
                pltpu.async_copy(btab.at[idx_v.at[j]],
                                 bias_v.at[pl.ds(j * 128, 128)], sem)
                for j in range(bias_groups)
            ]
            for h in handles:
                h.wait()
            pltpu.sync_copy(bias_v,
                            bout.at[pl.ds(lane * bias_groups * 128,
                                          bias_groups * 128)])


def _sc_gather(table, fidx, bidx, uidx, iidx, ubias, ibias, n_rows, B, D):
    n_steps = (n_rows // NW) // CHUNK
    bias_groups = (B // (NW // 2)) // 128
    mesh = plsc.VectorSubcoreMesh(core_axis_name="c", subcore_axis_name="s")
    body = functools.partial(_sc_gather_kernel, n_rows, n_steps, bias_groups, D)
    f = pl.kernel(
        body,
        out_type=(
            jax.ShapeDtypeStruct((n_rows, D), jnp.float32),
            jax.ShapeDtypeStruct((n_rows, D), jnp.float32),
            jax.ShapeDtypeStruct((B,), jnp.float32),
            jax.ShapeDtypeStruct((B,), jnp.float32),
        ),
        mesh=mesh,
        compiler_params=pltpu.CompilerParams(use_tc_tiling_on_sc=False),
        scratch_types=[
            pltpu.VMEM((GROUPS_PER_STEP, 128), jnp.int32),
            pltpu.VMEM((CHUNK, D), jnp.float32),
            pltpu.VMEM((bias_groups * 128,), jnp.float32),
            pltpu.SemaphoreType.DMA,
        ],
        name="sc_gather_rows_and_biases",
    )
    return f(table, fidx, bidx, uidx, iidx, ubias, ibias)


def _tc_pool_kernel(rf_ref, rb_ref, mf_ref, mb_ref, w_ref, ab_ref,
                    ub_ref, ib_ref, gb_ref, o_ref):
    w = w_ref[0]
    absum = jnp.sum(ab_ref[0])

    def pool(rows, mask):
        s = jnp.sum(rows * w[None, None, :], axis=-1) + absum
        s = jnp.where(mask, s, NEG_INF)
        m = jnp.max(s, axis=-1, keepdims=True)
        e = jnp.exp(s - m)
        d = jnp.sum(e, axis=-1, keepdims=True)
        p = e / d
        return jnp.sum(rows * p[:, :, None], axis=1)

    pu = pool(rf_ref[...], mf_ref[...] != 0)
    pi = pool(rb_ref[...], mb_ref[...] != 0)
    dot = jnp.sum(pu * pi, axis=-1, keepdims=True)
    o_ref[0] = dot + ub_ref[0] + ib_ref[0] + gb_ref[0, 0]


def _tc_pool(rows_f, rows_b, fidx, bidx, ub, ib, w, ab, gb, B, L, D, BE=256):
    nblk = B // BE
    grid = (nblk,)
    out = pl.pallas_call(
        _tc_pool_kernel,
        grid=grid,
        in_specs=[
            pl.BlockSpec((BE, L, D), lambda i: (i, 0, 0)),
            pl.BlockSpec((BE, L, D), lambda i: (i, 0, 0)),
            pl.BlockSpec((BE, L), lambda i: (i, 0)),
            pl.BlockSpec((BE, L), lambda i: (i, 0)),
            pl.BlockSpec((1, D), lambda i: (0, 0)),
            pl.BlockSpec((1, D), lambda i: (0, 0)),
            pl.BlockSpec((1, BE, 1), lambda i: (i, 0, 0)),
            pl.BlockSpec((1, BE, 1), lambda i: (i, 0, 0)),
            pl.BlockSpec((1, 1), lambda i: (0, 0)),
        ],
        out_specs=pl.BlockSpec((1, BE, 1), lambda i: (i, 0, 0)),
        out_shape=jax.ShapeDtypeStruct((nblk, BE, 1), jnp.float32),
    )(
        rows_f.reshape(B, L, D),
        rows_b.reshape(B, L, D),
        fidx, bidx,
        w.reshape(1, D), ab.reshape(1, D),
        ub.reshape(nblk, BE, 1), ib.reshape(nblk, BE, 1),
        gb.reshape(1, 1),
    )
    return out.reshape(B)


def kernel(user_idx, item_idx, fav_subjects, book_subjects, subj_emb,
           attn_weight, attn_bias, user_bias, item_bias, global_bias):
    B, L = fav_subjects.shape
    D = subj_emb.shape[1]
    n_rows = B * L

    fidx = fav_subjects.astype(jnp.int32).reshape(n_rows // 128, 128)
    bidx = book_subjects.astype(jnp.int32).reshape(n_rows // 128, 128)
    uidx = user_idx.astype(jnp.int32).reshape(B // 128, 128)
    iidx = item_idx.astype(jnp.int32).reshape(B // 128, 128)

    rows_f, rows_b, ub, ib = _sc_gather(
        subj_emb, fidx, bidx, uidx, iidx,
        user_bias.reshape(-1), item_bias.reshape(-1), n_rows, B, D)

    return _tc_pool(rows_f, rows_b, fav_subjects.astype(jnp.int32),
                    book_subjects.astype(jnp.int32), ub, ib,
                    attn_weight, attn_bias, global_bias, B, L, D)


# speedup vs baseline: 5.3859x; 1.1547x over previous
import functools

import jax
import jax.numpy as jnp
from jax import lax
from jax.experimental import pallas as pl
from jax.experimental.pallas import tpu as pltpu
from jax.experimental.pallas import tpu_sc as plsc

PAD_IDX = 0
NEG_INF = -1e9

NC = 2
NS = 16
NW = NC * NS

GROUPS_PER_STEP = 8
CHUNK = GROUPS_PER_STEP * 128


def _sc_gather_kernel(n_rows, n_steps, bias_groups, D,
                      table, fidx, bidx, uidx, iidx, ubias, ibias,
                      rows_f, rows_b, ub_out, ib_out,
                      idx_v, rows_v, bias_v, sem):
    wid = lax.axis_index("s") * NC + lax.axis_index("c")
    rows_per_w = n_rows // NW
    idxrows_per_w = rows_per_w // 128

    for idx_hbm, out_hbm in ((fidx, rows_f), (bidx, rows_b)):
        def step(s, carry, idx_hbm=idx_hbm, out_hbm=out_hbm):
            row_base = wid * rows_per_w + s * CHUNK
            irow = wid * idxrows_per_w + s * GROUPS_PER_STEP
            pltpu.sync_copy(idx_hbm.at[pl.ds(irow, GROUPS_PER_STEP)], idx_v)
            handles = [
                pltpu.async_copy(table.at[idx_v.at[j]],
                                 rows_v.at[pl.ds(j * 128, 128)], sem)
                for j in range(GROUPS_PER_STEP)
            ]
            for h in handles:
                h.wait()
            pltpu.sync_copy(rows_v, out_hbm.at[pl.ds(row_base, CHUNK)])
            return carry

        lax.fori_loop(0, n_steps, step, 0)

    half = NW // 2
    for active, bidx_hbm, btab, bout in ((wid < half, uidx, ubias, ub_out),
                                         (wid >= half, iidx, ibias, ib_out)):
        @pl.when(active)
        def _(bidx_hbm=bidx_hbm, btab=btab, bout=bout):
            lane = lax.rem(wid, half)
            pltpu.sync_copy(
                bidx_hbm.at[pl.ds(lane * bias_groups, bias_groups)],
                idx_v.at[pl.ds(0, bias_groups)])
            handles = [
                pltpu.async_copy(btab.at[idx_v.at[j]],
                                 bias_v.at[pl.ds(j * 128, 128)], sem)
                for j in range(bias_groups)
            ]
            for h in handles:
                h.wait()
            pltpu.sync_copy(bias_v,
                            bout.at[pl.ds(lane * bias_groups * 128,
                                          bias_groups * 128)])


def _sc_gather(table, fidx, bidx, uidx, iidx, ubias, ibias, n_rows, B, D):
    n_steps = (n_rows // NW) // CHUNK
    bias_groups = (B // (NW // 2)) // 128
    mesh = plsc.VectorSubcoreMesh(core_axis_name="c", subcore_axis_name="s")
    body = functools.partial(_sc_gather_kernel, n_rows, n_steps, bias_groups, D)
    f = pl.kernel(
        body,
        out_type=(
            jax.ShapeDtypeStruct((n_rows, D), jnp.float32),
            jax.ShapeDtypeStruct((n_rows, D), jnp.float32),
            jax.ShapeDtypeStruct((B,), jnp.float32),
            jax.ShapeDtypeStruct((B,), jnp.float32),
        ),
        mesh=mesh,
        compiler_params=pltpu.CompilerParams(use_tc_tiling_on_sc=False),
        scratch_types=[
            pltpu.VMEM((GROUPS_PER_STEP, 128), jnp.int32),
            pltpu.VMEM((CHUNK, D), jnp.float32),
            pltpu.VMEM((bias_groups * 128,), jnp.float32),
            pltpu.SemaphoreType.DMA,
        ],
        name="sc_gather_rows_and_biases",
    )
    return f(table, fidx, bidx, uidx, iidx, ubias, ibias)


def _dot(a, b):
    return jnp.dot(a, b, precision=jax.lax.Precision.HIGHEST,
                   preferred_element_type=jnp.float32)


def _tc_pool_kernel(rf_ref, rb_ref, mf_ref, mb_ref, wm_ref, e_ref,
                    r_ref, ab_ref, ub_ref, ib_ref, gb_ref, o_ref):
    absum = jnp.sum(ab_ref[...])

    def pool(rows, mask):
        s = _dot(rows, wm_ref[...]) + absum
        s = jnp.where(mask, s, NEG_INF)
        m = jnp.max(s, axis=-1, keepdims=True)
        e = jnp.exp(s - m)
        d = jnp.sum(e, axis=-1, keepdims=True)
        p = e / d
        pexp = _dot(p, e_ref[...])
        return _dot(pexp * rows, r_ref[...])

    pu = pool(rf_ref[...], mf_ref[...] != 0)
    pi = pool(rb_ref[...], mb_ref[...] != 0)
    dot = jnp.sum(pu * pi, axis=-1, keepdims=True)
    o_ref[0] = dot + ub_ref[0] + ib_ref[0] + gb_ref[0, 0]


def _tc_pool(rows_f, rows_b, fidx, bidx, ub, ib, w, ab, gb, B, L, D, BE=256):
    nblk = B // BE
    wmat = jnp.kron(jnp.eye(L, dtype=jnp.float32), w.reshape(D, 1))
    emat = jnp.kron(jnp.eye(L, dtype=jnp.float32),
                    jnp.ones((1, D), jnp.float32))
    rmat = jnp.kron(jnp.ones((L, 1), jnp.float32),
                    jnp.eye(D, dtype=jnp.float32))
    out = pl.pallas_call(
        _tc_pool_kernel,
        grid=(nblk,),
        in_specs=[
            pl.BlockSpec((BE, L * D), lambda i: (i, 0)),
            pl.BlockSpec((BE, L * D), lambda i: (i, 0)),
            pl.BlockSpec((BE, L), lambda i: (i, 0)),
            pl.BlockSpec((BE, L), lambda i: (i, 0)),
            pl.BlockSpec((L * D, L), lambda i: (0, 0)),
            pl.BlockSpec((L, L * D), lambda i: (0, 0)),
            pl.BlockSpec((L * D, D), lambda i: (0, 0)),
            pl.BlockSpec((1, D), lambda i: (0, 0)),
            pl.BlockSpec((1, BE, 1), lambda i: (i, 0, 0)),
            pl.BlockSpec((1, BE, 1), lambda i: (i, 0, 0)),
            pl.BlockSpec((1, 1), lambda i: (0, 0)),
        ],
        out_specs=pl.BlockSpec((1, BE, 1), lambda i: (i, 0, 0)),
        out_shape=jax.ShapeDtypeStruct((nblk, BE, 1), jnp.float32),
    )(
        rows_f.reshape(B, L * D),
        rows_b.reshape(B, L * D),
        fidx, bidx,
        wmat, emat, rmat, ab.reshape(1, D),
        ub.reshape(nblk, BE, 1), ib.reshape(nblk, BE, 1),
        gb.reshape(1, 1),
    )
    return out.reshape(B)


def kernel(user_idx, item_idx, fav_subjects, book_subjects, subj_emb,
           attn_weight, attn_bias, user_bias, item_bias, global_bias):
    B, L = fav_subjects.shape
    D = subj_emb.shape[1]
    n_rows = B * L

    fidx = fav_subjects.astype(jnp.int32).reshape(n_rows // 128, 128)
    bidx = book_subjects.astype(jnp.int32).reshape(n_rows // 128, 128)
    uidx = user_idx.astype(jnp.int32).reshape(B // 128, 128)
    iidx = item_idx.astype(jnp.int32).reshape(B // 128, 128)

    rows_f, rows_b, ub, ib = _sc_gather(
        subj_emb, fidx, bidx, uidx, iidx,
        user_bias.reshape(-1), item_bias.reshape(-1), n_rows, B, D)

    return _tc_pool(rows_f, rows_b, fav_subjects.astype(jnp.int32),
                    book_subjects.astype(jnp.int32), ub, ib,
                    attn_weight, attn_bias, global_bias, B, L, D)


# speedup vs baseline: 9.1100x; 1.6914x over previous
import functools

import jax
import jax.numpy as jnp
from jax import lax
from jax.experimental import pallas as pl
from jax.experimental.pallas import tpu as pltpu
from jax.experimental.pallas import tpu_sc as plsc

PAD_IDX = 0
NEG_INF = -1e9

NC = 2
NS = 16
NW = NC * NS

GROUPS_PER_STEP = 8
CHUNK = GROUPS_PER_STEP * 128


def _sc_gather_kernel(n_rows, n_steps, bias_groups, D,
                      table, fidx, bidx, uidx, iidx, ubias, ibias,
                      rows_f, rows_b, ub_out, ib_out,
                      idx_v, rows_v, bias_v, sem):
    wid = lax.axis_index("s") * NC + lax.axis_index("c")
    rows_per_w = n_rows // NW
    idxrows_per_w = rows_per_w // 128

    for idx_hbm, out_hbm in ((fidx, rows_f), (bidx, rows_b)):
        def step(s, carry, idx_hbm=idx_hbm, out_hbm=out_hbm):
            row_base = wid * rows_per_w + s * CHUNK
            irow = wid * idxrows_per_w + s * GROUPS_PER_STEP
            pltpu.sync_copy(idx_hbm.at[pl.ds(irow, GROUPS_PER_STEP)], idx_v)
            handles = [
                pltpu.async_copy(table.at[idx_v.at[j]],
                                 rows_v.at[pl.ds(j * 128, 128)], sem)
                for j in range(GROUPS_PER_STEP)
            ]
            for h in handles:
                h.wait()
            pltpu.sync_copy(rows_v, out_hbm.at[pl.ds(row_base, CHUNK)])
            return carry

        lax.fori_loop(0, n_steps, step, 0)

    half = NW // 2
    for active, bidx_hbm, btab, bout in ((wid < half, uidx, ubias, ub_out),
                                         (wid >= half, iidx, ibias, ib_out)):
        @pl.when(active)
        def _(bidx_hbm=bidx_hbm, btab=btab, bout=bout):
            lane = lax.rem(wid, half)
            pltpu.sync_copy(
                bidx_hbm.at[pl.ds(lane * bias_groups, bias_groups)],
                idx_v.at[pl.ds(0, bias_groups)])
            handles = [
                pltpu.async_copy(btab.at[idx_v.at[j]],
                                 bias_v.at[pl.ds(j * 128, 128)], sem)
                for j in range(bias_groups)
            ]
            for h in handles:
                h.wait()
            pltpu.sync_copy(bias_v,
                            bout.at[pl.ds(lane * bias_groups * 128,
                                          bias_groups * 128)])


def _sc_gather(table, fidx, bidx, uidx, iidx, ubias, ibias, n_rows, B, D):
    n_steps = (n_rows // NW) // CHUNK
    bias_groups = (B // (NW // 2)) // 128
    mesh = plsc.VectorSubcoreMesh(core_axis_name="c", subcore_axis_name="s")
    body = functools.partial(_sc_gather_kernel, n_rows, n_steps, bias_groups, D)
    f = pl.kernel(
        body,
        out_type=(
            jax.ShapeDtypeStruct((n_rows, D), jnp.float32),
            jax.ShapeDtypeStruct((n_rows, D), jnp.float32),
            jax.ShapeDtypeStruct((B,), jnp.float32),
            jax.ShapeDtypeStruct((B,), jnp.float32),
        ),
        mesh=mesh,
        compiler_params=pltpu.CompilerParams(use_tc_tiling_on_sc=False),
        scratch_types=[
            pltpu.VMEM((GROUPS_PER_STEP, 128), jnp.int32),
            pltpu.VMEM((CHUNK, D), jnp.float32),
            pltpu.VMEM((bias_groups * 128,), jnp.float32),
            pltpu.SemaphoreType.DMA,
        ],
        name="sc_gather_rows_and_biases",
    )
    return f(table, fidx, bidx, uidx, iidx, ubias, ibias)


def _dot(a, b):
    return jnp.dot(a, b, precision=jax.lax.Precision.DEFAULT,
                   preferred_element_type=jnp.float32)


def _tc_pool_kernel(rf_ref, rb_ref, mf_ref, mb_ref, wm_ref, e_ref,
                    r_ref, ab_ref, ub_ref, ib_ref, gb_ref, o_ref):
    absum = jnp.sum(ab_ref[...])

    def pool(rows, mask):
        s = _dot(rows, wm_ref[...]) + absum
        s = jnp.where(mask, s, NEG_INF)
        m = jnp.max(s, axis=-1, keepdims=True)
        e = jnp.exp(s - m)
        d = jnp.sum(e, axis=-1, keepdims=True)
        p = e / d
        pexp = _dot(p, e_ref[...])
        return _dot(pexp * rows, r_ref[...])

    pu = pool(rf_ref[...], mf_ref[...] != 0)
    pi = pool(rb_ref[...], mb_ref[...] != 0)
    dot = jnp.sum(pu * pi, axis=-1, keepdims=True)
    o_ref[0] = dot + ub_ref[0] + ib_ref[0] + gb_ref[0, 0]


def _tc_pool(rows_f, rows_b, fidx, bidx, ub, ib, w, ab, gb, B, L, D, BE=256):
    nblk = B // BE
    wmat = jnp.kron(jnp.eye(L, dtype=jnp.float32), w.reshape(D, 1))
    emat = jnp.kron(jnp.eye(L, dtype=jnp.float32),
                    jnp.ones((1, D), jnp.float32))
    rmat = jnp.kron(jnp.ones((L, 1), jnp.float32),
                    jnp.eye(D, dtype=jnp.float32))
    out = pl.pallas_call(
        _tc_pool_kernel,
        grid=(nblk,),
        in_specs=[
            pl.BlockSpec((BE, L * D), lambda i: (i, 0)),
            pl.BlockSpec((BE, L * D), lambda i: (i, 0)),
            pl.BlockSpec((BE, L), lambda i: (i, 0)),
            pl.BlockSpec((BE, L), lambda i: (i, 0)),
            pl.BlockSpec((L * D, L), lambda i: (0, 0)),
            pl.BlockSpec((L, L * D), lambda i: (0, 0)),
            pl.BlockSpec((L * D, D), lambda i: (0, 0)),
            pl.BlockSpec((1, D), lambda i: (0, 0)),
            pl.BlockSpec((1, BE, 1), lambda i: (i, 0, 0)),
            pl.BlockSpec((1, BE, 1), lambda i: (i, 0, 0)),
            pl.BlockSpec((1, 1), lambda i: (0, 0)),
        ],
        out_specs=pl.BlockSpec((1, BE, 1), lambda i: (i, 0, 0)),
        out_shape=jax.ShapeDtypeStruct((nblk, BE, 1), jnp.float32),
    )(
        rows_f.reshape(B, L * D),
        rows_b.reshape(B, L * D),
        fidx, bidx,
        wmat, emat, rmat, ab.reshape(1, D),
        ub.reshape(nblk, BE, 1), ib.reshape(nblk, BE, 1),
        gb.reshape(1, 1),
    )
    return out.reshape(B)


def kernel(user_idx, item_idx, fav_subjects, book_subjects, subj_emb,
           attn_weight, attn_bias, user_bias, item_bias, global_bias):
    B, L = fav_subjects.shape
    D = subj_emb.shape[1]
    n_rows = B * L

    fidx = fav_subjects.astype(jnp.int32).reshape(n_rows // 128, 128)
    bidx = book_subjects.astype(jnp.int32).reshape(n_rows // 128, 128)
    uidx = user_idx.astype(jnp.int32).reshape(B // 128, 128)
    iidx = item_idx.astype(jnp.int32).reshape(B // 128, 128)

    rows_f, rows_b, ub, ib = _sc_gather(
        subj_emb, fidx, bidx, uidx, iidx,
        user_bias.reshape(-1), item_bias.reshape(-1), n_rows, B, D)

    return _tc_pool(rows_f, rows_b, fav_subjects.astype(jnp.int32),
                    book_subjects.astype(jnp.int32), ub, ib,
                    attn_weight, attn_bias, global_bias, B, L, D)


# speedup vs baseline: 9.7458x; 1.0698x over previous
import functools

import jax
import jax.numpy as jnp
from jax import lax
from jax.experimental import pallas as pl
from jax.experimental.pallas import tpu as pltpu
from jax.experimental.pallas import tpu_sc as plsc

PAD_IDX = 0
NEG_INF = -1e9

NC = 2
NS = 16
NW = NC * NS

GROUPS_PER_STEP = 8
CHUNK = GROUPS_PER_STEP * 128


def _sc_gather_kernel(n_rows, n_steps, bias_groups, D,
                      table, fidx, bidx, uidx, iidx, ubias, ibias,
                      rows_f, rows_b, ub_out, ib_out,
                      idx_v, rows_v0, rows_v1, bias_v, sem0, sem1):
    wid = lax.axis_index("s") * NC + lax.axis_index("c")
    rows_per_w = n_rows // NW
    idxrows_per_w = rows_per_w // 128
    bufs = (rows_v0, rows_v1)
    sems = (sem0, sem1)

    def fire(k, p):
        for j in range(GROUPS_PER_STEP):
            pltpu.async_copy(table.at[idx_v.at[k * GROUPS_PER_STEP + j]],
                             bufs[p].at[pl.ds(j * 128, 128)], sems[p])

    def drain(p):
        for j in range(GROUPS_PER_STEP):
            pltpu.make_async_copy(table.at[idx_v.at[j]],
                                  bufs[p].at[pl.ds(j * 128, 128)],
                                  sems[p]).wait()

    for idx_hbm, out_hbm in ((fidx, rows_f), (bidx, rows_b)):
        pltpu.sync_copy(
            idx_hbm.at[pl.ds(wid * idxrows_per_w, idxrows_per_w)], idx_v)
        fire(0, 0)

        def step2(m, carry, out_hbm=out_hbm):
            for j in range(2):
                k = 2 * m + j
                drain(j)
                fire(k + 1, 1 - j)
                pltpu.sync_copy(
                    bufs[j],
                    out_hbm.at[pl.ds(wid * rows_per_w + k * CHUNK, CHUNK)])
            return carry

        lax.fori_loop(0, (n_steps - 1) // 2, step2, 0)
        drain((n_steps - 1) % 2)
        pltpu.sync_copy(
            bufs[(n_steps - 1) % 2],
            out_hbm.at[pl.ds(wid * rows_per_w + (n_steps - 1) * CHUNK,
                             CHUNK)])

    half = NW // 2
    for active, bidx_hbm, btab, bout in ((wid < half, uidx, ubias, ub_out),
                                         (wid >= half, iidx, ibias, ib_out)):
        @pl.when(active)
        def _(bidx_hbm=bidx_hbm, btab=btab, bout=bout):
            lane = lax.rem(wid, half)
            pltpu.sync_copy(
                bidx_hbm.at[pl.ds(lane * bias_groups, bias_groups)],
                idx_v.at[pl.ds(0, bias_groups)])
            handles = [
                pltpu.async_copy(btab.at[idx_v.at[j]],
                                 bias_v.at[pl.ds(j * 128, 128)], sem0)
                for j in range(bias_groups)
            ]
            for h in handles:
                h.wait()
            pltpu.sync_copy(bias_v,
                            bout.at[pl.ds(lane * bias_groups * 128,
                                          bias_groups * 128)])


def _sc_gather(table, fidx, bidx, uidx, iidx, ubias, ibias, n_rows, B, D):
    n_steps = (n_rows // NW) // CHUNK
    bias_groups = (B // (NW // 2)) // 128
    mesh = plsc.VectorSubcoreMesh(core_axis_name="c", subcore_axis_name="s")
    body = functools.partial(_sc_gather_kernel, n_rows, n_steps, bias_groups, D)
    f = pl.kernel(
        body,
        out_type=(
            jax.ShapeDtypeStruct((n_rows, D), jnp.float32),
            jax.ShapeDtypeStruct((n_rows, D), jnp.float32),
            jax.ShapeDtypeStruct((B,), jnp.float32),
            jax.ShapeDtypeStruct((B,), jnp.float32),
        ),
        mesh=mesh,
        compiler_params=pltpu.CompilerParams(use_tc_tiling_on_sc=False),
        scratch_types=[
            pltpu.VMEM(((n_rows // NW) // 128, 128), jnp.int32),
            pltpu.VMEM((CHUNK, D), jnp.float32),
            pltpu.VMEM((CHUNK, D), jnp.float32),
            pltpu.VMEM((bias_groups * 128,), jnp.float32),
            pltpu.SemaphoreType.DMA,
            pltpu.SemaphoreType.DMA,
        ],
        name="sc_gather_rows_and_biases",
    )
    return f(table, fidx, bidx, uidx, iidx, ubias, ibias)


def _dot(a, b):
    return jnp.dot(a, b, precision=jax.lax.Precision.DEFAULT,
                   preferred_element_type=jnp.float32)


def _tc_pool_kernel(rf_ref, rb_ref, mf_ref, mb_ref, wm_ref, e_ref,
                    r_ref, ab_ref, ub_ref, ib_ref, gb_ref, o_ref):
    absum = jnp.sum(ab_ref[...])

    def pool(rows, mask):
        s = _dot(rows, wm_ref[...]) + absum
        s = jnp.where(mask, s, NEG_INF)
        m = jnp.max(s, axis=-1, keepdims=True)
        e = jnp.exp(s - m)
        d = jnp.sum(e, axis=-1, keepdims=True)
        p = e / d
        pexp = _dot(p, e_ref[...])
        return _dot(pexp * rows, r_ref[...])

    pu = pool(rf_ref[...], mf_ref[...] != 0)
    pi = pool(rb_ref[...], mb_ref[...] != 0)
    dot = jnp.sum(pu * pi, axis=-1, keepdims=True)
    o_ref[0] = dot + ub_ref[0] + ib_ref[0] + gb_ref[0, 0]


def _tc_pool(rows_f, rows_b, fidx, bidx, ub, ib, w, ab, gb, B, L, D, BE=256):
    nblk = B // BE
    wmat = jnp.kron(jnp.eye(L, dtype=jnp.float32), w.reshape(D, 1))
    emat = jnp.kron(jnp.eye(L, dtype=jnp.float32),
                    jnp.ones((1, D), jnp.float32))
    rmat = jnp.kron(jnp.ones((L, 1), jnp.float32),
                    jnp.eye(D, dtype=jnp.float32))
    out = pl.pallas_call(
        _tc_pool_kernel,
        grid=(nblk,),
        in_specs=[
            pl.BlockSpec((BE, L * D), lambda i: (i, 0)),
            pl.BlockSpec((BE, L * D), lambda i: (i, 0)),
            pl.BlockSpec((BE, L), lambda i: (i, 0)),
            pl.BlockSpec((BE, L), lambda i: (i, 0)),
            pl.BlockSpec((L * D, L), lambda i: (0, 0)),
            pl.BlockSpec((L, L * D), lambda i: (0, 0)),
            pl.BlockSpec((L * D, D), lambda i: (0, 0)),
            pl.BlockSpec((1, D), lambda i: (0, 0)),
            pl.BlockSpec((1, BE, 1), lambda i: (i, 0, 0)),
            pl.BlockSpec((1, BE, 1), lambda i: (i, 0, 0)),
            pl.BlockSpec((1, 1), lambda i: (0, 0)),
        ],
        out_specs=pl.BlockSpec((1, BE, 1), lambda i: (i, 0, 0)),
        out_shape=jax.ShapeDtypeStruct((nblk, BE, 1), jnp.float32),
    )(
        rows_f.reshape(B, L * D),
        rows_b.reshape(B, L * D),
        fidx, bidx,
        wmat, emat, rmat, ab.reshape(1, D),
        ub.reshape(nblk, BE, 1), ib.reshape(nblk, BE, 1),
        gb.reshape(1, 1),
    )
    return out.reshape(B)


def kernel(user_idx, item_idx, fav_subjects, book_subjects, subj_emb,
           attn_weight, attn_bias, user_bias, item_bias, global_bias):
    B, L = fav_subjects.shape
    D = subj_emb.shape[1]
    n_rows = B * L

    fidx = fav_subjects.astype(jnp.int32).reshape(n_rows // 128, 128)
    bidx = book_subjects.astype(jnp.int32).reshape(n_rows // 128, 128)
    uidx = user_idx.astype(jnp.int32).reshape(B // 128, 128)
    iidx = item_idx.astype(jnp.int32).reshape(B // 128, 128)

    rows_f, rows_b, ub, ib = _sc_gather(
        subj_emb, fidx, bidx, uidx, iidx,
        user_bias.reshape(-1), item_bias.reshape(-1), n_rows, B, D)

    return _tc_pool(rows_f, rows_b, fav_subjects.astype(jnp.int32),
                    book_subjects.astype(jnp.int32), ub, ib,
                    attn_weight, attn_bias, global_bias, B, L, D)
